# baseline (speedup 1.0000x reference)
import math

import jax
import jax.numpy as jnp
from jax.experimental import pallas as pl
from jax.experimental.pallas import tpu as pltpu

NUM_BUCKETS = 32
MAX_DISTANCE = 128
NUM_HEADS = 16
Q_LEN = 2048
K_LEN = 2048
LV = 4352
LG = 4096
ROWS = 128


def _bias_kernel(table_ref, out_ref, g_ref):
    h = pl.program_id(0)
    g = pl.program_id(1)

    @pl.when(g == 0)
    def _build():
        m = jax.lax.broadcasted_iota(jnp.int32, (1, LV), 1)
        rel = jnp.maximum((Q_LEN - 1) - m, 0)
        relf = jnp.maximum(rel.astype(jnp.float32), 16.0)
        large = 16 + (
            jnp.log(relf / 16.0) * (16.0 / math.log(MAX_DISTANCE / 16.0))
        ).astype(jnp.int32)
        bucket = jnp.where(rel < 16, rel, jnp.minimum(large, NUM_BUCKETS - 1))

        lane = jax.lax.broadcasted_iota(jnp.int32, (NUM_BUCKETS, NUM_HEADS), 1)
        tcol = jnp.where(lane == h, table_ref[...], 0.0).sum(
            axis=1, keepdims=True
        )
        b_ids = jax.lax.broadcasted_iota(jnp.int32, (NUM_BUCKETS, LV), 0)
        onehot = (bucket == b_ids).astype(jnp.float32)
        v = jax.lax.dot_general(
            tcol,
            onehot,
            (((0,), (0,)), ((), ())),
            preferred_element_type=jnp.float32,
        )

        wide = jnp.broadcast_to(v, (ROWS, LV))
        shifted = pltpu.roll(wide, LV - 127, 1, stride=1, stride_axis=0)
        g_ref[...] = shifted[:, :LG]

    out_ref[0, 0, :, :] = g_ref[:, pl.ds(128 * (15 - g), K_LEN)]


def kernel(query_length, key_length, relative_attention_bias):
    del query_length, key_length
    out = pl.pallas_call(
        _bias_kernel,
        grid=(NUM_HEADS, Q_LEN // ROWS),
        in_specs=[
            pl.BlockSpec((NUM_BUCKETS, NUM_HEADS), lambda h, g: (0, 0)),
        ],
        out_specs=pl.BlockSpec(
            (1, 1, ROWS, K_LEN), lambda h, g: (0, h, g, 0)
        ),
        out_shape=jax.ShapeDtypeStruct(
            (1, NUM_HEADS, Q_LEN, K_LEN), jnp.float32
        ),
        scratch_shapes=[pltpu.VMEM((ROWS, LG), jnp.float32)],
        compiler_params=pltpu.CompilerParams(
            dimension_semantics=("parallel", "arbitrary"),
        ),
    )(relative_attention_bias)
    return out


# speedup vs baseline: 1.3737x; 1.3737x over previous
import math

import jax
import jax.numpy as jnp
from jax.experimental import pallas as pl
from jax.experimental.pallas import tpu as pltpu

NUM_BUCKETS = 32
MAX_DISTANCE = 128
NUM_HEADS = 16
Q_LEN = 2048
K_LEN = 2048
LV = 4352
LG = 4096
ROWS = 128
BLK_I = 256


def _bias_kernel(table_ref, out_ref, g_ref):
    h = pl.program_id(0)
    g = pl.program_id(1)

    @pl.when(g == 0)
    def _build():
        m = jax.lax.broadcasted_iota(jnp.int32, (1, LV), 1)
        rel = jnp.maximum((Q_LEN - 1) - m, 0)
        relf = jnp.maximum(rel.astype(jnp.float32), 16.0)
        large = 16 + (
            jnp.log(relf / 16.0) * (16.0 / math.log(MAX_DISTANCE / 16.0))
        ).astype(jnp.int32)
        bucket = jnp.where(rel < 16, rel, jnp.minimum(large, NUM_BUCKETS - 1))

        lane = jax.lax.broadcasted_iota(jnp.int32, (NUM_BUCKETS, NUM_HEADS), 1)
        tcol = jnp.where(lane == h, table_ref[...], 0.0).sum(
            axis=1, keepdims=True
        )
        b_ids = jax.lax.broadcasted_iota(jnp.int32, (NUM_BUCKETS, LV), 0)
        onehot = (bucket == b_ids).astype(jnp.float32)
        v = jax.lax.dot_general(
            tcol,
            onehot,
            (((0,), (0,)), ((), ())),
            preferred_element_type=jnp.float32,
        )

        wide = jnp.broadcast_to(v, (ROWS, LV))
        shifted = pltpu.roll(wide, LV - 127, 1, stride=1, stride_axis=0)
        g_ref[...] = shifted[:, :LG]

    out_ref[0, 0, 0:128, :] = g_ref[:, pl.ds(128 * (15 - 2 * g), K_LEN)]
    out_ref[0, 0, 128:256, :] = g_ref[:, pl.ds(128 * (14 - 2 * g), K_LEN)]


def kernel(query_length, key_length, relative_attention_bias):
    del query_length, key_length
    out = pl.pallas_call(
        _bias_kernel,
        grid=(NUM_HEADS, Q_LEN // BLK_I),
        in_specs=[
            pl.BlockSpec((NUM_BUCKETS, NUM_HEADS), lambda h, g: (0, 0)),
        ],
        out_specs=pl.BlockSpec(
            (1, 1, BLK_I, K_LEN), lambda h, g: (0, h, g, 0)
        ),
        out_shape=jax.ShapeDtypeStruct(
            (1, NUM_HEADS, Q_LEN, K_LEN), jnp.float32
        ),
        scratch_shapes=[pltpu.VMEM((ROWS, LG), jnp.float32)],
        compiler_params=pltpu.CompilerParams(
            dimension_semantics=("parallel", "arbitrary"),
        ),
    )(relative_attention_bias)
    return out


# speedup vs baseline: 1.6184x; 1.1781x over previous
import math

import jax
import jax.numpy as jnp
from jax.experimental import pallas as pl
from jax.experimental.pallas import tpu as pltpu

NUM_BUCKETS = 32
MAX_DISTANCE = 128
NUM_HEADS = 16
Q_LEN = 2048
K_LEN = 2048
LV = 4352
LG = 4096
ROWS = 128
BLK_I = 512


def _bias_kernel(table_ref, out_ref, g_ref):
    h = pl.program_id(0)
    g = pl.program_id(1)

    @pl.when(g == 0)
    def _build():
        m = jax.lax.broadcasted_iota(jnp.int32, (1, LV), 1)
        rel = jnp.maximum((Q_LEN - 1) - m, 0)
        relf = jnp.maximum(rel.astype(jnp.float32), 16.0)
        large = 16 + (
            jnp.log(relf / 16.0) * (16.0 / math.log(MAX_DISTANCE / 16.0))
        ).astype(jnp.int32)
        bucket = jnp.where(rel < 16, rel, jnp.minimum(large, NUM_BUCKETS - 1))

        lane = jax.lax.broadcasted_iota(jnp.int32, (NUM_BUCKETS, NUM_HEADS), 1)
        tcol = jnp.where(lane == h, table_ref[...], 0.0).sum(
            axis=1, keepdims=True
        )
        b_ids = jax.lax.broadcasted_iota(jnp.int32, (NUM_BUCKETS, LV), 0)
        onehot = (bucket == b_ids).astype(jnp.float32)
        v = jax.lax.dot_general(
            tcol,
            onehot,
            (((0,), (0,)), ((), ())),
            preferred_element_type=jnp.float32,
        )

        wide = jnp.broadcast_to(v, (ROWS, LV))
        shifted = pltpu.roll(wide, LV - 127, 1, stride=1, stride_axis=0)
        g_ref[...] = shifted[:, :LG]

    ngrp = BLK_I // ROWS
    for k in range(ngrp):
        out_ref[0, 0, ROWS * k : ROWS * (k + 1), :] = g_ref[
            :, pl.ds(128 * (15 - ngrp * g - k), K_LEN)
        ]


def kernel(query_length, key_length, relative_attention_bias):
    del query_length, key_length
    out = pl.pallas_call(
        _bias_kernel,
        grid=(NUM_HEADS, Q_LEN // BLK_I),
        in_specs=[
            pl.BlockSpec((NUM_BUCKETS, NUM_HEADS), lambda h, g: (0, 0)),
        ],
        out_specs=pl.BlockSpec(
            (1, 1, BLK_I, K_LEN), lambda h, g: (0, h, g, 0)
        ),
        out_shape=jax.ShapeDtypeStruct(
            (1, NUM_HEADS, Q_LEN, K_LEN), jnp.float32
        ),
        scratch_shapes=[pltpu.VMEM((ROWS, LG), jnp.float32)],
        compiler_params=pltpu.CompilerParams(
            dimension_semantics=("parallel", "arbitrary"),
        ),
    )(relative_attention_bias)
    return out


# speedup vs baseline: 1.7846x; 1.1027x over previous
import math

import jax
import jax.numpy as jnp
from jax.experimental import pallas as pl
from jax.experimental.pallas import tpu as pltpu

NUM_BUCKETS = 32
MAX_DISTANCE = 128
NUM_HEADS = 16
Q_LEN = 2048
K_LEN = 2048
LV = 4352
LG = 4096
ROWS = 128
BLK_I = 1024


def _bias_kernel(table_ref, out_ref, g_ref):
    h = pl.program_id(0)
    g = pl.program_id(1)

    @pl.when(g == 0)
    def _build():
        m = jax.lax.broadcasted_iota(jnp.int32, (1, LV), 1)
        rel = jnp.maximum((Q_LEN - 1) - m, 0)
        relf = jnp.maximum(rel.astype(jnp.float32), 16.0)
        large = 16 + (
            jnp.log(relf / 16.0) * (16.0 / math.log(MAX_DISTANCE / 16.0))
        ).astype(jnp.int32)
        bucket = jnp.where(rel < 16, rel, jnp.minimum(large, NUM_BUCKETS - 1))

        lane = jax.lax.broadcasted_iota(jnp.int32, (NUM_BUCKETS, NUM_HEADS), 1)
        tcol = jnp.where(lane == h, table_ref[...], 0.0).sum(
            axis=1, keepdims=True
        )
        b_ids = jax.lax.broadcasted_iota(jnp.int32, (NUM_BUCKETS, LV), 0)
        onehot = (bucket == b_ids).astype(jnp.float32)
        v = jax.lax.dot_general(
            tcol,
            onehot,
            (((0,), (0,)), ((), ())),
            preferred_element_type=jnp.float32,
        )

        wide = jnp.broadcast_to(v, (ROWS, LV))
        shifted = pltpu.roll(wide, LV - 127, 1, stride=1, stride_axis=0)
        g_ref[...] = shifted[:, :LG]

    ngrp = BLK_I // ROWS
    for k in range(ngrp):
        out_ref[0, 0, ROWS * k : ROWS * (k + 1), :] = g_ref[
            :, pl.ds(128 * (15 - ngrp * g - k), K_LEN)
        ]


def kernel(query_length, key_length, relative_attention_bias):
    del query_length, key_length
    out = pl.pallas_call(
        _bias_kernel,
        grid=(NUM_HEADS, Q_LEN // BLK_I),
        in_specs=[
            pl.BlockSpec((NUM_BUCKETS, NUM_HEADS), lambda h, g: (0, 0)),
        ],
        out_specs=pl.BlockSpec(
            (1, 1, BLK_I, K_LEN), lambda h, g: (0, h, g, 0)
        ),
        out_shape=jax.ShapeDtypeStruct(
            (1, NUM_HEADS, Q_LEN, K_LEN), jnp.float32
        ),
        scratch_shapes=[pltpu.VMEM((ROWS, LG), jnp.float32)],
        compiler_params=pltpu.CompilerParams(
            dimension_semantics=("parallel", "arbitrary"),
        ),
    )(relative_attention_bias)
    return out
